# baseline (speedup 1.0000x reference)
import functools

import jax
import jax.numpy as jnp
from jax import lax
from jax.experimental import pallas as pl
from jax.experimental.pallas import tpu as pltpu
from jax.experimental.pallas import tpu_sc as plsc

V = 100000
H = 8
D = 16
B = 16384
OUT = H * D
NC = 2
NS = 16
NW = NC * NS
LOGW = 15
W = 1 << LOGW
VMAIN = 99968
NWIN = 5
ICH = 4096
NICH = B // ICH
NVREG = ICH // 16

_mesh = plsc.VectorSubcoreMesh(core_axis_name="c", subcore_axis_name="s")


def _win_size(k):
    if k < 3:
        return W
    return 1664 if k == 3 else 128


@functools.partial(
    pl.kernel,
    mesh=_mesh,
    out_type=jax.ShapeDtypeStruct((OUT, B), jnp.float32),
    compiler_params=pltpu.CompilerParams(
        use_tc_tiling_on_sc=True, needs_layout_passes=False),
    scratch_types=[
        pltpu.VMEM((ICH,), jnp.int32),
        pltpu.VMEM((B + 16,), jnp.int32),
        pltpu.VMEM((16,), jnp.int32),
        pltpu.VMEM((W,), jnp.float32),
        pltpu.VMEM((W,), jnp.float32),
        pltpu.VMEM((B,), jnp.float32),
        pltpu.VMEM((B,), jnp.float32),
        pltpu.SemaphoreType.DMA,
        pltpu.SemaphoreType.DMA,
    ],
)
def _sc_embed(tbl_hbm, tail_hbm, idx_hbm, out_hbm, ichunk, plist, cnt,
              winbuf0, winbuf1, recip, col, sem0, sem1):
    wid = lax.axis_index("s") * NC + lax.axis_index("c")
    h = wid // 4
    cg0 = (wid % 4) * 4
    lanes = lax.iota(jnp.int32, 16)
    cnt[...] = jnp.zeros((16,), jnp.int32)

    for ch in range(NICH):
        pltpu.sync_copy(idx_hbm.at[pl.ds(ch * ICH, ICH)], ichunk)

        def hist(i, _):
            v = ichunk[pl.ds(i * 16, 16)]
            k = jnp.where(v >= VMAIN, 4, v >> LOGW)
            plsc.addupdate_scatter(cnt, [k], jnp.ones((16,), jnp.int32))
            return 0

        lax.fori_loop(0, NVREG, hist, 0)
    cnts = cnt[...]
    seg_off = plsc.cumsum(cnts) - cnts

    offs = seg_off
    for ch in range(NICH):
        pltpu.sync_copy(idx_hbm.at[pl.ds(ch * ICH, ICH)], ichunk)

        def fill(i, offs):
            v = ichunk[pl.ds(i * 16, 16)]
            kv = jnp.where(v >= VMAIN, 4, v >> LOGW)
            lv = jnp.where(v >= VMAIN, v - VMAIN, v & (W - 1))
            pk = (lv << 14) | (ch * ICH + i * 16 + lanes)
            for k in range(NWIN):
                m = kv == k
                n = plsc.all_reduce_population_count(m)
                plsc.store_compressed(plist.at[pl.ds(offs[k], 16)], pk, mask=m)
                offs = offs + jnp.where(lanes == k, n, 0)
            return offs

        offs = lax.fori_loop(0, NVREG, fill, offs)

    chans = [D] + [cg0 + ci for ci in range(4)]
    steps = [(r, k) for r in range(5) for k in range(NWIN)]
    bufs = (winbuf0, winbuf1)
    sems = (sem0, sem1)

    def start(s, slot):
        c, k = chans[steps[s][0]], steps[s][1]
        sz = _win_size(k)
        if k == 4:
            return pltpu.async_copy(tail_hbm.at[c, h],
                                    bufs[slot].at[pl.ds(0, sz)], sems[slot])
        return pltpu.async_copy(tbl_hbm.at[c, h, pl.ds(k * W, sz)],
                                bufs[slot].at[pl.ds(0, sz)], sems[slot])

    def serve(s, slot):
        ri, k = steps[s]
        s_k = seg_off[k]
        n_k = cnts[k]
        win = bufs[slot]

        def body(j2, _):
            for u in range(4):
                j = j2 * 4 + u
                off = s_k + j * 16
                p = plist[pl.ds(off, 16)]
                msk = (j * 16 + lanes) < n_k
                b = p & (B - 1)
                lv = p >> 14
                x = plsc.load_gather(win, [lv], mask=msk)
                if ri == 0:
                    plsc.store_scatter(recip, [b], 1.0 / x, mask=msk)
                else:
                    r = plsc.load_gather(recip, [b], mask=msk)
                    plsc.store_scatter(col, [b], x * r, mask=msk)
            return 0

        lax.fori_loop(0, (n_k + 63) >> 6, body, 0)

    pend = [None, None]
    pend[0] = start(0, 0)
    for s in range(len(steps)):
        slot = s % 2
        if s + 1 < len(steps):
            pend[1 - slot] = start(s + 1, 1 - slot)
        pend[slot].wait()
        serve(s, slot)
        ri, k = steps[s]
        if k == NWIN - 1 and ri > 0:
            pltpu.sync_copy(col, out_hbm.at[h * D + chans[ri]])


def kernel(memory, nodes):
    tbl = memory.transpose(2, 1, 0)
    tail = jnp.pad(memory[VMAIN:], ((0, 128 - (V - VMAIN)), (0, 0), (0, 0)))
    tail = tail.transpose(2, 1, 0)
    out_t = _sc_embed(tbl, tail, nodes.astype(jnp.int32))
    return out_t.T


# speedup vs baseline: 1.0266x; 1.0266x over previous
import functools

import jax
import jax.numpy as jnp
from jax import lax
from jax.experimental import pallas as pl
from jax.experimental.pallas import tpu as pltpu
from jax.experimental.pallas import tpu_sc as plsc

V = 100000
H = 8
D = 16
B = 16384
OUT = H * D
NC = 2
NS = 16
NW = NC * NS
LOGW = 15
W = 1 << LOGW
VMAIN = 99968
NWIN = 5
ICH = 4096
NICH = B // ICH
NVREG = ICH // 16

_mesh = plsc.VectorSubcoreMesh(core_axis_name="c", subcore_axis_name="s")


def _win_size(k):
    if k < 3:
        return W
    return 1664 if k == 3 else 128


@functools.partial(
    pl.kernel,
    mesh=_mesh,
    out_type=jax.ShapeDtypeStruct((OUT, B), jnp.float32),
    compiler_params=pltpu.CompilerParams(
        use_tc_tiling_on_sc=True, needs_layout_passes=False),
    scratch_types=[
        pltpu.VMEM((ICH,), jnp.int32),
        pltpu.VMEM((B + 16,), jnp.int32),
        pltpu.VMEM((16,), jnp.int32),
        pltpu.VMEM((W,), jnp.float32),
        pltpu.VMEM((W,), jnp.float32),
        pltpu.VMEM((B,), jnp.float32),
        pltpu.VMEM((B,), jnp.float32),
        pltpu.SemaphoreType.DMA,
        pltpu.SemaphoreType.DMA,
    ],
)
def _sc_embed(tbl_hbm, tail_hbm, idx_hbm, out_hbm, ichunk, plist, cnt,
              winbuf0, winbuf1, recip, col, sem0, sem1):
    wid = lax.axis_index("s") * NC + lax.axis_index("c")
    h = wid // 4
    cg0 = (wid % 4) * 4
    lanes = lax.iota(jnp.int32, 16)
    cnt[...] = jnp.zeros((16,), jnp.int32)

    for ch in range(NICH):
        pltpu.sync_copy(idx_hbm.at[pl.ds(ch * ICH, ICH)], ichunk)

        def hist(i, _):
            v = ichunk[pl.ds(i * 16, 16)]
            k = jnp.where(v >= VMAIN, 4, v >> LOGW)
            plsc.addupdate_scatter(cnt, [k], jnp.ones((16,), jnp.int32))
            return 0

        lax.fori_loop(0, NVREG, hist, 0)
    cnts = cnt[...]
    seg_off = plsc.cumsum(cnts) - cnts

    offs = seg_off
    for ch in range(NICH):
        pltpu.sync_copy(idx_hbm.at[pl.ds(ch * ICH, ICH)], ichunk)

        def fill(i, offs):
            v = ichunk[pl.ds(i * 16, 16)]
            kv = jnp.where(v >= VMAIN, 4, v >> LOGW)
            lv = jnp.where(v >= VMAIN, v - VMAIN, v & (W - 1))
            pk = (lv << 14) | (ch * ICH + i * 16 + lanes)
            for k in range(NWIN):
                m = kv == k
                n = plsc.all_reduce_population_count(m)
                plsc.store_compressed(plist.at[pl.ds(offs[k], 16)], pk, mask=m)
                offs = offs + jnp.where(lanes == k, n, 0)
            return offs

        offs = lax.fori_loop(0, NVREG, fill, offs)

    chans = [D] + [cg0 + ci for ci in range(4)]
    steps = [(r, k) for r in range(5) for k in range(NWIN)]
    bufs = (winbuf0, winbuf1)
    sems = (sem0, sem1)

    def start(s, slot):
        c, k = chans[steps[s][0]], steps[s][1]
        sz = _win_size(k)
        if k == 4:
            return pltpu.async_copy(tail_hbm.at[c, h],
                                    bufs[slot].at[pl.ds(0, sz)], sems[slot])
        return pltpu.async_copy(tbl_hbm.at[c, h, pl.ds(k * W, sz)],
                                bufs[slot].at[pl.ds(0, sz)], sems[slot])

    def serve(s, slot):
        ri, k = steps[s]
        s_k = seg_off[k]
        n_k = cnts[k]
        win = bufs[slot]

        def full(j, _):
            off = s_k + j * 16
            p = plist[pl.ds(off, 16)]
            b = p & (B - 1)
            lv = p >> 14
            x = plsc.load_gather(win, [lv])
            if ri == 0:
                plsc.store_scatter(recip, [b], 1.0 / x)
            else:
                r = plsc.load_gather(recip, [b])
                plsc.store_scatter(col, [b], x * r)
            return 0

        nfull = n_k >> 4
        lax.fori_loop(0, nfull, full, 0)

        @pl.when(n_k & 15 != 0)
        def _tail():
            off = s_k + nfull * 16
            p = plist[pl.ds(off, 16)]
            msk = lanes < (n_k & 15)
            b = p & (B - 1)
            lv = p >> 14
            x = plsc.load_gather(win, [lv], mask=msk)
            if ri == 0:
                plsc.store_scatter(recip, [b], 1.0 / x, mask=msk)
            else:
                r = plsc.load_gather(recip, [b], mask=msk)
                plsc.store_scatter(col, [b], x * r, mask=msk)

    pend = [None, None]
    pend[0] = start(0, 0)
    for s in range(len(steps)):
        slot = s % 2
        if s + 1 < len(steps):
            pend[1 - slot] = start(s + 1, 1 - slot)
        pend[slot].wait()
        serve(s, slot)
        ri, k = steps[s]
        if k == NWIN - 1 and ri > 0:
            pltpu.sync_copy(col, out_hbm.at[h * D + chans[ri]])


def kernel(memory, nodes):
    tbl = memory.transpose(2, 1, 0)
    tail = jnp.pad(memory[VMAIN:], ((0, 128 - (V - VMAIN)), (0, 0), (0, 0)))
    tail = tail.transpose(2, 1, 0)
    out_t = _sc_embed(tbl, tail, nodes.astype(jnp.int32))
    return out_t.T


# speedup vs baseline: 1.4714x; 1.4332x over previous
import functools

import jax
import jax.numpy as jnp
from jax import lax
from jax.experimental import pallas as pl
from jax.experimental.pallas import tpu as pltpu
from jax.experimental.pallas import tpu_sc as plsc

V = 100000
H = 8
D = 16
B = 16384
OUT = H * D
NC = 2
NS = 16
NW = NC * NS
LOGW = 15
W = 1 << LOGW
VMAIN = 99968
NWIN = 5
ICH = 4096
NICH = B // ICH
NVREG = ICH // 16

_mesh = plsc.VectorSubcoreMesh(core_axis_name="c", subcore_axis_name="s")


def _win_size(k):
    if k < 3:
        return W
    return 1664 if k == 3 else 128


@functools.partial(
    pl.kernel,
    mesh=_mesh,
    out_type=jax.ShapeDtypeStruct((OUT, B), jnp.float32),
    compiler_params=pltpu.CompilerParams(
        use_tc_tiling_on_sc=True, needs_layout_passes=False),
    scratch_types=[
        pltpu.VMEM((ICH,), jnp.int32),
        pltpu.VMEM((B + 16,), jnp.int32),
        pltpu.VMEM((16,), jnp.int32),
        pltpu.VMEM((W,), jnp.float32),
        pltpu.VMEM((W,), jnp.float32),
        pltpu.VMEM((B,), jnp.float32),
        pltpu.VMEM((B,), jnp.float32),
        pltpu.SemaphoreType.DMA,
        pltpu.SemaphoreType.DMA,
    ],
)
def _sc_embed(tbl_hbm, tail_hbm, idx_hbm, out_hbm, ichunk, plist, cnt,
              winbuf0, winbuf1, recip, col, sem0, sem1):
    wid = lax.axis_index("s") * NC + lax.axis_index("c")
    h = wid // 4
    cg0 = (wid % 4) * 4
    lanes = lax.iota(jnp.int32, 16)
    cnt[...] = jnp.zeros((16,), jnp.int32)

    for ch in range(NICH):
        pltpu.sync_copy(idx_hbm.at[pl.ds(ch * ICH, ICH)], ichunk)

        def hist(i, _):
            v = ichunk[pl.ds(i * 16, 16)]
            k = jnp.where(v >= VMAIN, 4, v >> LOGW)
            plsc.addupdate_scatter(cnt, [k], jnp.ones((16,), jnp.int32))
            return 0

        lax.fori_loop(0, NVREG, hist, 0)
    cnts = cnt[...]
    seg_off = plsc.cumsum(cnts) - cnts

    offs = seg_off
    for ch in range(NICH):
        pltpu.sync_copy(idx_hbm.at[pl.ds(ch * ICH, ICH)], ichunk)

        def fill(i, offs):
            v = ichunk[pl.ds(i * 16, 16)]
            kv = jnp.where(v >= VMAIN, 4, v >> LOGW)
            lv = jnp.where(v >= VMAIN, v - VMAIN, v & (W - 1))
            pk = (lv << 14) | (ch * ICH + i * 16 + lanes)
            for k in range(NWIN):
                m = kv == k
                n = plsc.all_reduce_population_count(m)
                plsc.store_compressed(plist.at[pl.ds(offs[k], 16)], pk, mask=m)
                offs = offs + jnp.where(lanes == k, n, 0)
            return offs

        offs = lax.fori_loop(0, NVREG, fill, offs)

    chans = [D] + [cg0 + ci for ci in range(4)]
    steps = [(r, k) for r in range(5) for k in range(NWIN)]
    bufs = (winbuf0, winbuf1)
    sems = (sem0, sem1)

    def start(s, slot):
        c, k = chans[steps[s][0]], steps[s][1]
        sz = _win_size(k)
        if k == 4:
            return pltpu.async_copy(tail_hbm.at[c, h],
                                    bufs[slot].at[pl.ds(0, sz)], sems[slot])
        return pltpu.async_copy(tbl_hbm.at[c, h, pl.ds(k * W, sz)],
                                bufs[slot].at[pl.ds(0, sz)], sems[slot])

    def serve(s, slot):
        ri, k = steps[s]
        s_k = seg_off[k]
        n_k = cnts[k]
        win = bufs[slot]

        nfull = n_k >> 4

        @plsc.parallel_loop(0, nfull, 1, unroll=4)
        def _full(j):
            off = s_k + j * 16
            p = plist[pl.ds(off, 16)]
            b = p & (B - 1)
            lv = p >> 14
            x = plsc.load_gather(win, [lv])
            if ri == 0:
                plsc.store_scatter(recip, [b], 1.0 / x)
            else:
                r = plsc.load_gather(recip, [b])
                plsc.store_scatter(col, [b], x * r)

        @pl.when(n_k & 15 != 0)
        def _tail():
            off = s_k + nfull * 16
            p = plist[pl.ds(off, 16)]
            msk = lanes < (n_k & 15)
            b = p & (B - 1)
            lv = p >> 14
            x = plsc.load_gather(win, [lv], mask=msk)
            if ri == 0:
                plsc.store_scatter(recip, [b], 1.0 / x, mask=msk)
            else:
                r = plsc.load_gather(recip, [b], mask=msk)
                plsc.store_scatter(col, [b], x * r, mask=msk)

    pend = [None, None]
    pend[0] = start(0, 0)
    for s in range(len(steps)):
        slot = s % 2
        if s + 1 < len(steps):
            pend[1 - slot] = start(s + 1, 1 - slot)
        pend[slot].wait()
        serve(s, slot)
        ri, k = steps[s]
        if k == NWIN - 1 and ri > 0:
            pltpu.sync_copy(col, out_hbm.at[h * D + chans[ri]])


def kernel(memory, nodes):
    tbl = memory.transpose(2, 1, 0)
    tail = jnp.pad(memory[VMAIN:], ((0, 128 - (V - VMAIN)), (0, 0), (0, 0)))
    tail = tail.transpose(2, 1, 0)
    out_t = _sc_embed(tbl, tail, nodes.astype(jnp.int32))
    return out_t.T
